# initial kernel scaffold (unmeasured)
import jax
import jax.numpy as jnp
from jax import lax
from jax.experimental import pallas as pl
from jax.experimental.pallas import tpu as pltpu

N_DEV = 8
T = 512
V_SHARD = 8192

_ANY = pltpu.ANY if hasattr(pltpu, "ANY") else pltpu.MemorySpace.ANY


def kernel(x, W):
    t, v_shard = x.shape[0], W.shape[1]
    assert (t, v_shard) == (T, V_SHARD), (t, v_shard)

    logits = x @ W
    m = logits.max(axis=1, keepdims=True)
    e = jnp.exp(logits - m)
    s = e.sum(axis=1, keepdims=True)
    stats = jnp.concatenate([m, s], axis=1)

    def body(e_ref, stats_ref, out_ref,
             stats_buf, pbuf,
             local_sem, stats_send_sems, stats_recv_sems,
             ring_send_sems, ring_recv_sems):
        my = lax.axis_index("i")

        own_copy = pltpu.make_async_copy(stats_ref, stats_buf.at[my], local_sem)
        own_copy.start()

        sends = []
        for k in range(1, N_DEV):
            dst_dev = (my + k) % N_DEV
            r = pltpu.make_async_remote_copy(
                src_ref=stats_ref,
                dst_ref=stats_buf.at[my],
                send_sem=stats_send_sems.at[k],
                recv_sem=stats_recv_sems.at[k],
                device_id=(dst_dev,),
                device_id_type=pl.DeviceIdType.MESH,
            )
            r.start()
            sends.append(r)

        own_copy.wait()
        for k in range(1, N_DEV):
            rcv = pltpu.make_async_remote_copy(
                src_ref=stats_ref,
                dst_ref=stats_buf.at[k],
                send_sem=stats_send_sems.at[k],
                recv_sem=stats_recv_sems.at[k],
                device_id=(my,),
                device_id_type=pl.DeviceIdType.MESH,
            )
            rcv.wait_recv()
        for r in sends:
            r.wait_send()

        M = stats_buf[0, :, 0:1]
        for d in range(1, N_DEV):
            M = jnp.maximum(M, stats_buf[d, :, 0:1])
        S = jnp.zeros_like(M)
        for d in range(N_DEV):
            S = S + stats_buf[d, :, 1:2] * jnp.exp(stats_buf[d, :, 0:1] - M)
        scale = jnp.exp(stats_ref[:, 0:1] - M) / S
        pbuf[...] = e_ref[...] * scale

        out_slice = out_ref.at[:, pl.ds(my * V_SHARD, V_SHARD)]
        store = pltpu.make_async_copy(pbuf, out_slice, local_sem)
        store.start()
        store.wait()

        right = (my + 1) % N_DEV
        for h in range(N_DEV - 1):
            origin = (my - h + N_DEV) % N_DEV
            sl = pl.ds(origin * V_SHARD, V_SHARD)
            rdma = pltpu.make_async_remote_copy(
                src_ref=out_ref.at[:, sl],
                dst_ref=out_ref.at[:, sl],
                send_sem=ring_send_sems.at[h],
                recv_sem=ring_recv_sems.at[h],
                device_id=(right,),
                device_id_type=pl.DeviceIdType.MESH,
            )
            rdma.start()
            rdma.wait()

    return pl.pallas_call(
        body,
        out_shape=jax.ShapeDtypeStruct((T, N_DEV * V_SHARD), jnp.float32),
        in_specs=[
            pl.BlockSpec(memory_space=pltpu.VMEM),
            pl.BlockSpec(memory_space=pltpu.VMEM),
        ],
        out_specs=pl.BlockSpec(memory_space=_ANY),
        scratch_shapes=[
            pltpu.VMEM((N_DEV, T, 2), jnp.float32),
            pltpu.VMEM((T, V_SHARD), jnp.float32),
            pltpu.SemaphoreType.DMA,
            pltpu.SemaphoreType.DMA((N_DEV,)),
            pltpu.SemaphoreType.DMA((N_DEV,)),
            pltpu.SemaphoreType.DMA((N_DEV - 1,)),
            pltpu.SemaphoreType.DMA((N_DEV - 1,)),
        ],
        compiler_params=pltpu.CompilerParams(collective_id=0),
    )(e, stats)


# baseline (device time: 1415625 ns/iter reference)
import jax
import jax.numpy as jnp
from jax import lax
from jax.experimental import pallas as pl
from jax.experimental.pallas import tpu as pltpu

N_DEV = 8
T = 512
V_SHARD = 8192

_ANY = pl.ANY


def kernel(x, W):
    t, v_shard = x.shape[0], W.shape[1]
    assert (t, v_shard) == (T, V_SHARD), (t, v_shard)

    logits = x @ W
    m = logits.max(axis=1, keepdims=True)
    e = jnp.exp(logits - m)
    s = e.sum(axis=1, keepdims=True)
    stats = jnp.concatenate([m, s], axis=1)

    def body(e_ref, stats_ref, out_ref,
             stats_buf, pbuf,
             local_sem, stats_send_sems, stats_recv_sems,
             ring_send_sems, ring_recv_sems):
        my = lax.axis_index("i")

        barrier_sem = pltpu.get_barrier_semaphore()
        for k in range(1, N_DEV):
            pl.semaphore_signal(
                barrier_sem, inc=1,
                device_id=((my + k) % N_DEV,),
                device_id_type=pl.DeviceIdType.MESH,
            )
        pl.semaphore_wait(barrier_sem, N_DEV - 1)

        own_copy = pltpu.make_async_copy(stats_ref, stats_buf.at[my], local_sem)
        own_copy.start()

        sends = []
        for k in range(1, N_DEV):
            dst_dev = (my + k) % N_DEV
            r = pltpu.make_async_remote_copy(
                src_ref=stats_ref,
                dst_ref=stats_buf.at[my],
                send_sem=stats_send_sems.at[k],
                recv_sem=stats_recv_sems.at[k],
                device_id=(dst_dev,),
                device_id_type=pl.DeviceIdType.MESH,
            )
            r.start()
            sends.append(r)

        own_copy.wait()
        for k in range(1, N_DEV):
            rcv = pltpu.make_async_remote_copy(
                src_ref=stats_ref,
                dst_ref=stats_buf.at[k],
                send_sem=stats_send_sems.at[k],
                recv_sem=stats_recv_sems.at[k],
                device_id=(my,),
                device_id_type=pl.DeviceIdType.MESH,
            )
            rcv.wait_recv()
        for r in sends:
            r.wait_send()

        M = stats_buf[0, :, 0:1]
        for d in range(1, N_DEV):
            M = jnp.maximum(M, stats_buf[d, :, 0:1])
        S = jnp.zeros_like(M)
        for d in range(N_DEV):
            S = S + stats_buf[d, :, 1:2] * jnp.exp(stats_buf[d, :, 0:1] - M)
        scale = jnp.exp(stats_ref[:, 0:1] - M) / S
        pbuf[...] = e_ref[...] * scale

        out_slice = out_ref.at[:, pl.ds(my * V_SHARD, V_SHARD)]
        store = pltpu.make_async_copy(pbuf, out_slice, local_sem)
        store.start()
        store.wait()

        right = (my + 1) % N_DEV
        for h in range(N_DEV - 1):
            origin = (my - h + N_DEV) % N_DEV
            sl = pl.ds(origin * V_SHARD, V_SHARD)
            rdma = pltpu.make_async_remote_copy(
                src_ref=out_ref.at[:, sl],
                dst_ref=out_ref.at[:, sl],
                send_sem=ring_send_sems.at[h],
                recv_sem=ring_recv_sems.at[h],
                device_id=(right,),
                device_id_type=pl.DeviceIdType.MESH,
            )
            rdma.start()
            rdma.wait()

    return pl.pallas_call(
        body,
        out_shape=jax.ShapeDtypeStruct((T, N_DEV * V_SHARD), jnp.float32),
        in_specs=[
            pl.BlockSpec(memory_space=pltpu.VMEM),
            pl.BlockSpec(memory_space=pltpu.VMEM),
        ],
        out_specs=pl.BlockSpec(memory_space=_ANY),
        scratch_shapes=[
            pltpu.VMEM((N_DEV, T, 2), jnp.float32),
            pltpu.VMEM((T, V_SHARD), jnp.float32),
            pltpu.SemaphoreType.DMA,
            pltpu.SemaphoreType.DMA((N_DEV,)),
            pltpu.SemaphoreType.DMA((N_DEV,)),
            pltpu.SemaphoreType.DMA((N_DEV - 1,)),
            pltpu.SemaphoreType.DMA((N_DEV - 1,)),
        ],
        compiler_params=pltpu.CompilerParams(collective_id=0),
    )(e, stats)
